# baseline (speedup 1.0000x reference)
import functools

import jax

jax.config.update("jax_enable_x64", True)

import jax.numpy as jnp
from jax import lax
from jax.experimental import pallas as pl
from jax.experimental.pallas import tpu as pltpu
from jax.experimental.pallas import tpu_sc as plsc

NC, NS, LANES = 2, 16, 16
NW = NC * NS

PAD_W, PAD_C, PAD_T = 100016, 144, 64

B, L = 4096, 200
LB = 40
NLC = L // LB


def _sc_gather_call(s3, c3, t3, wt_pad, ct_pad, tt_pad):
    mesh = plsc.VectorSubcoreMesh(core_axis_name="c", subcore_axis_name="s")

    @functools.partial(
        pl.kernel,
        out_type=(
            jax.ShapeDtypeStruct(s3.shape, jnp.int32),
            jax.ShapeDtypeStruct(c3.shape, jnp.int32),
            jax.ShapeDtypeStruct(t3.shape, jnp.int32),
        ),
        mesh=mesh,
        scratch_types=[
            pltpu.VMEM((PAD_W,), jnp.int32),
            pltpu.VMEM((PAD_C,), jnp.int32),
            pltpu.VMEM((PAD_T,), jnp.int32),
            pltpu.VMEM((LB, 128), jnp.int32),
            pltpu.VMEM((LB, 128), jnp.int32),
        ],
        compiler_params=pltpu.CompilerParams(
            needs_layout_passes=False, use_tc_tiling_on_sc=True),
    )
    def run(s_hbm, c_hbm, t_hbm, wt_hbm, ct_hbm, tt_hbm,
            so_hbm, co_hbm, to_hbm,
            wt_v, ct_v, tt_v, in_v, out_v):
        wid = lax.axis_index("s") * NC + lax.axis_index("c")
        b0 = wid * jnp.int32(128)

        pltpu.sync_copy(wt_hbm, wt_v)
        pltpu.sync_copy(ct_hbm, ct_v)
        pltpu.sync_copy(tt_hbm, tt_v)

        def phase(in_hbm, out_hbm, table_v, wdim):
            def plane(w, _):
                def lchunk(li, _):
                    l0 = li * jnp.int32(LB)
                    pltpu.sync_copy(
                        in_hbm.at[w, pl.ds(l0, LB), pl.ds(b0, 128)], in_v)

                    def row(r, _):
                        for cc in range(128 // LANES):
                            v = in_v[r, pl.ds(jnp.int32(cc * LANES), LANES)]
                            out_v[r, pl.ds(jnp.int32(cc * LANES), LANES)] = (
                                plsc.load_gather(table_v, [v]))
                        return jnp.int32(0)

                    lax.fori_loop(jnp.int32(0), jnp.int32(LB), row,
                                  jnp.int32(0))
                    pltpu.sync_copy(
                        out_v, out_hbm.at[w, pl.ds(l0, LB), pl.ds(b0, 128)])
                    return jnp.int32(0)

                lax.fori_loop(jnp.int32(0), jnp.int32(NLC), lchunk,
                              jnp.int32(0))
                return jnp.int32(0)

            lax.fori_loop(jnp.int32(0), jnp.int32(wdim), plane, jnp.int32(0))

        phase(s_hbm, so_hbm, wt_v, 1)
        phase(c_hbm, co_hbm, ct_v, c3.shape[0])
        phase(t_hbm, to_hbm, tt_v, 1)

    return run(s3, c3, t3, wt_pad, ct_pad, tt_pad)


def kernel(sentence_tensor, char_tensor, tag_string_tensor,
           word_table, char_table, tag_table):
    s3 = sentence_tensor.astype(jnp.int32).transpose(1, 0).reshape(1, L, B)
    c3 = char_tensor.astype(jnp.int32).transpose(2, 1, 0)
    t3 = tag_string_tensor.astype(jnp.int32).transpose(1, 0).reshape(1, L, B)

    def pad_table(tb, pad_len):
        out = jnp.zeros((pad_len,), jnp.int32)
        return out.at[: tb.shape[0]].set(tb.astype(jnp.int32))

    wt_pad = pad_table(word_table, PAD_W)
    ct_pad = pad_table(char_table, PAD_C)
    tt_pad = pad_table(tag_table, PAD_T)

    so, co, to = _sc_gather_call(s3, c3, t3, wt_pad, ct_pad, tt_pad)

    def widen(x32):
        return x32.astype(jnp.uint32).astype(jnp.uint64).astype(jnp.int64)

    return (
        widen(so.reshape(L, B).transpose(1, 0)),
        widen(co.transpose(2, 1, 0)),
        widen(to.reshape(L, B).transpose(1, 0)),
    )


# speedup vs baseline: 1.0347x; 1.0347x over previous
import functools

import jax

jax.config.update("jax_enable_x64", True)

import jax.numpy as jnp
from jax import lax
from jax.experimental import pallas as pl
from jax.experimental.pallas import tpu as pltpu
from jax.experimental.pallas import tpu_sc as plsc

NC, NS, LANES = 2, 16, 16
NW = NC * NS

PAD_W, PAD_C, PAD_T = 100016, 144, 64

B, L = 4096, 200
LB = 40
NLC = L // LB


def _sc_gather_call(s3, c3, t3, wt_pad, ct_pad, tt_pad):
    mesh = plsc.VectorSubcoreMesh(core_axis_name="c", subcore_axis_name="s")

    @functools.partial(
        pl.kernel,
        out_type=(
            jax.ShapeDtypeStruct(s3.shape, jnp.uint32),
            jax.ShapeDtypeStruct(c3.shape, jnp.uint32),
            jax.ShapeDtypeStruct(t3.shape, jnp.uint32),
        ),
        mesh=mesh,
        scratch_types=[
            pltpu.VMEM((PAD_W,), jnp.int32),
            pltpu.VMEM((PAD_C,), jnp.int32),
            pltpu.VMEM((PAD_T,), jnp.int32),
            pltpu.VMEM((LB, 128), jnp.uint32),
            pltpu.VMEM((LB, 128), jnp.uint32),
        ],
        compiler_params=pltpu.CompilerParams(
            needs_layout_passes=False, use_tc_tiling_on_sc=True),
    )
    def run(s_hbm, c_hbm, t_hbm, wt_hbm, ct_hbm, tt_hbm,
            so_hbm, co_hbm, to_hbm,
            wt_v, ct_v, tt_v, in_v, out_v):
        wid = lax.axis_index("s") * NC + lax.axis_index("c")
        b0 = wid * jnp.int32(128)

        pltpu.sync_copy(wt_hbm, wt_v)
        pltpu.sync_copy(ct_hbm, ct_v)
        pltpu.sync_copy(tt_hbm, tt_v)

        def phase(in_hbm, out_hbm, table_v, wdim):
            def plane(w, _):
                def lchunk(li, _):
                    l0 = li * jnp.int32(LB)
                    pltpu.sync_copy(
                        in_hbm.at[w, pl.ds(l0, LB), pl.ds(b0, 128)], in_v)

                    def row(r, _):
                        for cc in range(128 // LANES):
                            v = in_v[r, pl.ds(jnp.int32(cc * LANES), LANES)]
                            g = plsc.load_gather(
                                table_v, [plsc.bitcast(v, jnp.int32)])
                            out_v[r, pl.ds(jnp.int32(cc * LANES), LANES)] = (
                                plsc.bitcast(g, jnp.uint32))
                        return jnp.int32(0)

                    lax.fori_loop(jnp.int32(0), jnp.int32(LB), row,
                                  jnp.int32(0))
                    pltpu.sync_copy(
                        out_v, out_hbm.at[w, pl.ds(l0, LB), pl.ds(b0, 128)])
                    return jnp.int32(0)

                lax.fori_loop(jnp.int32(0), jnp.int32(NLC), lchunk,
                              jnp.int32(0))
                return jnp.int32(0)

            lax.fori_loop(jnp.int32(0), jnp.int32(wdim), plane, jnp.int32(0))

        phase(s_hbm, so_hbm, wt_v, 1)
        phase(c_hbm, co_hbm, ct_v, c3.shape[0])
        phase(t_hbm, to_hbm, tt_v, 1)

    return run(s3, c3, t3, wt_pad, ct_pad, tt_pad)


def kernel(sentence_tensor, char_tensor, tag_string_tensor,
           word_table, char_table, tag_table):
    s3 = sentence_tensor.astype(jnp.uint32).transpose(1, 0).reshape(1, L, B)
    c3 = char_tensor.astype(jnp.uint32).transpose(2, 1, 0)
    t3 = tag_string_tensor.astype(jnp.uint32).transpose(1, 0).reshape(1, L, B)

    def pad_table(tb, pad_len):
        out = jnp.zeros((pad_len,), jnp.int32)
        return out.at[: tb.shape[0]].set(tb.astype(jnp.int32))

    wt_pad = pad_table(word_table, PAD_W)
    ct_pad = pad_table(char_table, PAD_C)
    tt_pad = pad_table(tag_table, PAD_T)

    so, co, to = _sc_gather_call(s3, c3, t3, wt_pad, ct_pad, tt_pad)

    def widen(x32):
        return x32.astype(jnp.uint64).astype(jnp.int64)

    return (
        widen(so.reshape(L, B).transpose(1, 0)),
        widen(co.transpose(2, 1, 0)),
        widen(to.reshape(L, B).transpose(1, 0)),
    )


# speedup vs baseline: 1.0819x; 1.0456x over previous
import functools

import jax

jax.config.update("jax_enable_x64", True)

import jax.numpy as jnp
from jax import lax
from jax.experimental import pallas as pl
from jax.experimental.pallas import tpu as pltpu
from jax.experimental.pallas import tpu_sc as plsc

NC, NS, LANES = 2, 16, 16
NW = NC * NS

PAD_W, PAD_C, PAD_T = 100016, 144, 64

B, L = 4096, 200
LB = 40
NLC = L // LB


def _make_gather(n_in, table_pads, wdims):
    mesh = plsc.VectorSubcoreMesh(core_axis_name="c", subcore_axis_name="s")

    def run(*refs):
        ins = refs[:n_in]
        tables_hbm = refs[n_in:2 * n_in]
        outs = refs[2 * n_in:3 * n_in]
        tables_v = refs[3 * n_in:4 * n_in]
        in_v, out_v = refs[4 * n_in], refs[4 * n_in + 1]

        wid = lax.axis_index("s") * NC + lax.axis_index("c")
        b0 = wid * jnp.int32(128)

        for t_hbm, t_v in zip(tables_hbm, tables_v):
            pltpu.sync_copy(t_hbm, t_v)

        for in_hbm, out_hbm, table_v, wdim in zip(ins, outs, tables_v, wdims):
            def plane(w, _, in_hbm=in_hbm, out_hbm=out_hbm, table_v=table_v):
                def lchunk(li, _):
                    l0 = li * jnp.int32(LB)
                    pltpu.sync_copy(
                        in_hbm.at[w, pl.ds(l0, LB), pl.ds(b0, 128)], in_v)

                    def row(r, _):
                        for cc in range(128 // LANES):
                            v = in_v[r, pl.ds(jnp.int32(cc * LANES), LANES)]
                            g = plsc.load_gather(
                                table_v, [plsc.bitcast(v, jnp.int32)])
                            out_v[r, pl.ds(jnp.int32(cc * LANES), LANES)] = (
                                plsc.bitcast(g, jnp.uint32))
                        return jnp.int32(0)

                    lax.fori_loop(jnp.int32(0), jnp.int32(LB), row,
                                  jnp.int32(0))
                    pltpu.sync_copy(
                        out_v, out_hbm.at[w, pl.ds(l0, LB), pl.ds(b0, 128)])
                    return jnp.int32(0)

                lax.fori_loop(jnp.int32(0), jnp.int32(NLC), lchunk,
                              jnp.int32(0))
                return jnp.int32(0)

            lax.fori_loop(jnp.int32(0), jnp.int32(wdim), plane, jnp.int32(0))

    return functools.partial(
        pl.kernel,
        out_type=tuple(
            jax.ShapeDtypeStruct((w, L, B), jnp.uint32) for w in wdims),
        mesh=mesh,
        scratch_types=(
            [pltpu.VMEM((p,), jnp.int32) for p in table_pads]
            + [pltpu.VMEM((LB, 128), jnp.uint32),
               pltpu.VMEM((LB, 128), jnp.uint32)]
        ),
        compiler_params=pltpu.CompilerParams(needs_layout_passes=False),
    )(run)


def kernel(sentence_tensor, char_tensor, tag_string_tensor,
           word_table, char_table, tag_table):
    s3 = sentence_tensor.astype(jnp.uint32).transpose(1, 0).reshape(1, L, B)
    c3 = char_tensor.astype(jnp.uint32).transpose(2, 1, 0)
    t3 = tag_string_tensor.astype(jnp.uint32).transpose(1, 0).reshape(1, L, B)

    def pad_table(tb, pad_len):
        out = jnp.zeros((pad_len,), jnp.int32)
        return out.at[: tb.shape[0]].set(tb.astype(jnp.int32))

    wt_pad = pad_table(word_table, PAD_W)
    ct_pad = pad_table(char_table, PAD_C)
    tt_pad = pad_table(tag_table, PAD_T)

    so, to = _make_gather(2, (PAD_W, PAD_T), (1, 1))(s3, t3, wt_pad, tt_pad)
    (co,) = _make_gather(1, (PAD_C,), (c3.shape[0],))(c3, ct_pad)

    def widen(x32):
        return x32.astype(jnp.uint64).astype(jnp.int64)

    return (
        widen(so.reshape(L, B).transpose(1, 0)),
        widen(co.transpose(2, 1, 0)),
        widen(to.reshape(L, B).transpose(1, 0)),
    )


# speedup vs baseline: 1.1403x; 1.0539x over previous
import functools

import jax

jax.config.update("jax_enable_x64", True)

import jax.numpy as jnp
from jax import lax
from jax.experimental import pallas as pl
from jax.experimental.pallas import tpu as pltpu
from jax.experimental.pallas import tpu_sc as plsc

NC, NS, LANES = 2, 16, 16
NW = NC * NS

PAD_W, PAD_C, PAD_T = 100016, 144, 64

B, L = 4096, 200
LB = 40
NLC = L // LB


def _make_gather(n_in, table_pads, wdims):
    mesh = plsc.VectorSubcoreMesh(core_axis_name="c", subcore_axis_name="s")

    def run(*refs):
        ins = refs[:n_in]
        tables_hbm = refs[n_in:2 * n_in]
        outs = refs[2 * n_in:3 * n_in]
        tables_v = refs[3 * n_in:4 * n_in]
        in_b = refs[4 * n_in:4 * n_in + 2]
        out_b = refs[4 * n_in + 2:4 * n_in + 4]
        sin = refs[4 * n_in + 4:4 * n_in + 6]
        sout = refs[4 * n_in + 6:4 * n_in + 8]

        wid = lax.axis_index("s") * NC + lax.axis_index("c")
        b0 = wid * jnp.int32(128)

        for t_hbm, t_v in zip(tables_hbm, tables_v):
            pltpu.sync_copy(t_hbm, t_v)

        for in_hbm, out_hbm, table_v, wdim in zip(ins, outs, tables_v, wdims):
            def plane(w, _, in_hbm=in_hbm, out_hbm=out_hbm, table_v=table_v):
                def src(li):
                    return in_hbm.at[w, pl.ds(li * jnp.int32(LB), LB),
                                     pl.ds(b0, 128)]

                def dst(li):
                    return out_hbm.at[w, pl.ds(li * jnp.int32(LB), LB),
                                      pl.ds(b0, 128)]

                pltpu.async_copy(src(jnp.int32(0)), in_b[0], sin[0])
                for li in range(NLC):
                    p = li % 2
                    pltpu.make_async_copy(src(jnp.int32(li)), in_b[p],
                                          sin[p]).wait()
                    if li + 1 < NLC:
                        pltpu.async_copy(src(jnp.int32(li + 1)),
                                         in_b[1 - p], sin[1 - p])
                    if li >= 2:
                        pltpu.make_async_copy(out_b[p], dst(jnp.int32(li - 2)),
                                              sout[p]).wait()
                    in_v, out_v = in_b[p], out_b[p]

                    def row(r, _, in_v=in_v, out_v=out_v):
                        for cc in range(128 // LANES):
                            v = in_v[r, pl.ds(jnp.int32(cc * LANES), LANES)]
                            g = plsc.load_gather(
                                table_v, [plsc.bitcast(v, jnp.int32)])
                            out_v[r, pl.ds(jnp.int32(cc * LANES), LANES)] = (
                                plsc.bitcast(g, jnp.uint32))
                        return jnp.int32(0)

                    lax.fori_loop(jnp.int32(0), jnp.int32(LB), row,
                                  jnp.int32(0))
                    pltpu.async_copy(out_b[p], dst(jnp.int32(li)), sout[p])
                for li in (NLC - 2, NLC - 1):
                    p = li % 2
                    pltpu.make_async_copy(out_b[p], dst(jnp.int32(li)),
                                          sout[p]).wait()
                return jnp.int32(0)

            lax.fori_loop(jnp.int32(0), jnp.int32(wdim), plane, jnp.int32(0))

    return functools.partial(
        pl.kernel,
        out_type=tuple(
            jax.ShapeDtypeStruct((w, L, B), jnp.uint32) for w in wdims),
        mesh=mesh,
        scratch_types=(
            [pltpu.VMEM((p,), jnp.int32) for p in table_pads]
            + [pltpu.VMEM((LB, 128), jnp.uint32) for _ in range(4)]
            + [pltpu.SemaphoreType.DMA for _ in range(4)]
        ),
        compiler_params=pltpu.CompilerParams(needs_layout_passes=False),
    )(run)


def kernel(sentence_tensor, char_tensor, tag_string_tensor,
           word_table, char_table, tag_table):
    s3 = sentence_tensor.astype(jnp.uint32).transpose(1, 0).reshape(1, L, B)
    c3 = char_tensor.astype(jnp.uint32).transpose(2, 1, 0)
    t3 = tag_string_tensor.astype(jnp.uint32).transpose(1, 0).reshape(1, L, B)

    def pad_table(tb, pad_len):
        out = jnp.zeros((pad_len,), jnp.int32)
        return out.at[: tb.shape[0]].set(tb.astype(jnp.int32))

    wt_pad = pad_table(word_table, PAD_W)
    ct_pad = pad_table(char_table, PAD_C)
    tt_pad = pad_table(tag_table, PAD_T)

    so, to = _make_gather(2, (PAD_W, PAD_T), (1, 1))(s3, t3, wt_pad, tt_pad)
    (co,) = _make_gather(1, (PAD_C,), (c3.shape[0],))(c3, ct_pad)

    def widen(x32):
        return x32.astype(jnp.uint64).astype(jnp.int64)

    return (
        widen(so.reshape(L, B).transpose(1, 0)),
        widen(co.transpose(2, 1, 0)),
        widen(to.reshape(L, B).transpose(1, 0)),
    )
